# Initial kernel scaffold; baseline (speedup 1.0000x reference)
import jax
import jax.numpy as jnp
from jax.experimental import pallas as pl


def kernel(position_ids, weight):
    raise NotImplementedError("write your pallas kernel here")


# speedup vs baseline: 1.3799x; 1.3799x over previous
import functools

import jax
import jax.numpy as jnp
from jax import lax
from jax.experimental import pallas as pl
from jax.experimental.pallas import tpu as pltpu
from jax.experimental.pallas import tpu_sc as plsc

OFFSET = 2
HIDDEN = 2048
NC = 2
NS = 16
NW = NC * NS
CHUNK = 16


@functools.lru_cache(maxsize=None)
def _build(n_total):
    b_per_w = n_total // NW
    n_chunk = b_per_w // CHUNK
    mesh = plsc.VectorSubcoreMesh(core_axis_name="c", subcore_axis_name="s")

    @functools.partial(
        pl.kernel,
        out_type=jax.ShapeDtypeStruct((n_total, HIDDEN), jnp.float32),
        mesh=mesh,
        scratch_types=[
            pltpu.VMEM((n_chunk, CHUNK), jnp.int32),
            pltpu.VMEM((2, CHUNK, HIDDEN), jnp.float32),
            pltpu.SemaphoreType.DMA,
            pltpu.SemaphoreType.DMA,
            pltpu.SemaphoreType.DMA,
        ],
    )
    def gather_kernel(idx_hbm, table_hbm, out_hbm, idx_v, rows_v, gsem, s0, s1):
        wid = lax.axis_index("s") * NC + lax.axis_index("c")
        base = wid * b_per_w
        pltpu.sync_copy(idx_hbm.at[wid], idx_v)
        ssems = (s0, s1)
        for j in range(n_chunk):
            buf = j % 2
            ivec = idx_v[j] + OFFSET
            if j >= 2:
                pltpu.make_async_copy(
                    rows_v.at[buf], out_hbm.at[pl.ds(base, CHUNK)], ssems[buf]
                ).wait()
            pltpu.async_copy(table_hbm.at[ivec], rows_v.at[buf], gsem).wait()
            pltpu.async_copy(
                rows_v.at[buf],
                out_hbm.at[pl.ds(base + j * CHUNK, CHUNK)],
                ssems[buf],
            )
        for j in (n_chunk - 2, n_chunk - 1):
            buf = j % 2
            pltpu.make_async_copy(
                rows_v.at[buf], out_hbm.at[pl.ds(base, CHUNK)], ssems[buf]
            ).wait()

    return gather_kernel


def kernel(position_ids, weight):
    batch, seq = position_ids.shape
    ids = jnp.transpose(position_ids, (1, 0)).astype(jnp.int32)
    n_total = batch * seq
    idx = ids.reshape(NW, (n_total // NW) // CHUNK, CHUNK)
    out = _build(n_total)(idx, weight)
    return out.reshape(seq, batch, HIDDEN)


# speedup vs baseline: 1.4160x; 1.0262x over previous
import functools

import jax
import jax.numpy as jnp
from jax import lax
from jax.experimental import pallas as pl
from jax.experimental.pallas import tpu as pltpu
from jax.experimental.pallas import tpu_sc as plsc

OFFSET = 2
HIDDEN = 2048
NC = 2
NS = 16
NW = NC * NS
CHUNK = 16


@functools.lru_cache(maxsize=None)
def _build(n_total):
    b_per_w = n_total // NW
    n_chunk = b_per_w // CHUNK
    mesh = plsc.VectorSubcoreMesh(core_axis_name="c", subcore_axis_name="s")

    nbuf = 3

    @functools.partial(
        pl.kernel,
        out_type=jax.ShapeDtypeStruct((n_total, HIDDEN), jnp.float32),
        mesh=mesh,
        scratch_types=[
            pltpu.VMEM((n_chunk, CHUNK), jnp.int32),
            pltpu.VMEM((nbuf, CHUNK, HIDDEN), jnp.float32),
            [pltpu.SemaphoreType.DMA] * nbuf,
            [pltpu.SemaphoreType.DMA] * nbuf,
        ],
    )
    def gather_kernel(idx_hbm, table_hbm, out_hbm, idx_v, rows_v, gsems, ssems):
        wid = lax.axis_index("s") * NC + lax.axis_index("c")
        base = wid * b_per_w
        pltpu.sync_copy(idx_hbm.at[wid], idx_v)

        ghandles = [None] * n_chunk
        shandles = [None] * n_chunk

        def start_gather(j):
            buf = j % nbuf
            ivec = idx_v[j] + OFFSET
            ghandles[j] = pltpu.async_copy(
                table_hbm.at[ivec], rows_v.at[buf], gsems[buf]
            )

        for j in range(min(nbuf - 1, n_chunk)):
            start_gather(j)
        for j in range(n_chunk):
            buf = j % nbuf
            ghandles[j].wait()
            shandles[j] = pltpu.async_copy(
                rows_v.at[buf],
                out_hbm.at[pl.ds(base + j * CHUNK, CHUNK)],
                ssems[buf],
            )
            nj = j + nbuf - 1
            if nj < n_chunk:
                if nj >= nbuf:
                    shandles[nj - nbuf].wait()
                start_gather(nj)
        for j in range(n_chunk - nbuf, n_chunk):
            shandles[j].wait()

    return gather_kernel


def kernel(position_ids, weight):
    batch, seq = position_ids.shape
    ids = jnp.transpose(position_ids, (1, 0)).astype(jnp.int32)
    n_total = batch * seq
    idx = ids.reshape(NW, (n_total // NW) // CHUNK, CHUNK)
    out = _build(n_total)(idx, weight)
    return out.reshape(seq, batch, HIDDEN)


# speedup vs baseline: 2.9383x; 2.0750x over previous
import functools

import jax
import jax.numpy as jnp
from jax import lax
from jax.experimental import pallas as pl
from jax.experimental.pallas import tpu as pltpu
from jax.experimental.pallas import tpu_sc as plsc

OFFSET = 2
HIDDEN = 2048
NC = 2
NS = 16
NW = NC * NS
CHUNK = 16


@functools.lru_cache(maxsize=None)
def _build(n_total, batch, seq):
    b_per_w = n_total // NW
    n_chunk = b_per_w // CHUNK
    s_per_chunk = CHUNK // batch
    mesh = plsc.VectorSubcoreMesh(core_axis_name="c", subcore_axis_name="s")

    nbuf = 3

    @functools.partial(
        pl.kernel,
        out_type=jax.ShapeDtypeStruct((seq, batch, HIDDEN), jnp.float32),
        mesh=mesh,
        scratch_types=[
            pltpu.VMEM((n_chunk, CHUNK), jnp.int32),
            pltpu.VMEM((nbuf, CHUNK, HIDDEN), jnp.float32),
            [pltpu.SemaphoreType.DMA] * nbuf,
            [pltpu.SemaphoreType.DMA] * nbuf,
        ],
    )
    def gather_kernel(idx_hbm, table_hbm, out_hbm, idx_v, rows_v, gsems, ssems):
        wid = lax.axis_index("s") * NC + lax.axis_index("c")
        s_base = wid * (b_per_w // batch)
        pltpu.sync_copy(idx_hbm.at[wid], idx_v)

        ghandles = [None] * n_chunk
        shandles = [None] * n_chunk

        def start_gather(j):
            buf = j % nbuf
            ivec = idx_v[j] + OFFSET
            ghandles[j] = pltpu.async_copy(
                table_hbm.at[ivec], rows_v.at[buf], gsems[buf]
            )

        def start_store(j):
            buf = j % nbuf
            shandles[j] = [
                pltpu.async_copy(
                    rows_v.at[buf, pl.ds(i * batch, batch)],
                    out_hbm.at[s_base + j * s_per_chunk + i],
                    ssems[buf],
                )
                for i in range(s_per_chunk)
            ]

        for j in range(min(nbuf - 1, n_chunk)):
            start_gather(j)
        for j in range(n_chunk):
            ghandles[j].wait()
            start_store(j)
            nj = j + nbuf - 1
            if nj < n_chunk:
                if nj >= nbuf:
                    for h in shandles[nj - nbuf]:
                        h.wait()
                start_gather(nj)
        for j in range(n_chunk - nbuf, n_chunk):
            for h in shandles[j]:
                h.wait()

    return gather_kernel


def kernel(position_ids, weight):
    batch, seq = position_ids.shape
    ids = jnp.transpose(position_ids, (1, 0)).astype(jnp.int32)
    n_total = batch * seq
    idx = ids.reshape(NW, (n_total // NW) // CHUNK, CHUNK)
    return _build(n_total, batch, seq)(idx, weight)
